# baseline (speedup 1.0000x reference)
import functools

import jax
import jax.numpy as jnp
from jax.experimental import pallas as pl

NQ = 8
K = 1024
D = 64
BLK = 4096
NSPLIT = 1


def _rvq_kernel(x_ref, cb_ref, qout_ref, idx_ref, loss_ref, *, n_tokens):
    i = pl.program_id(0)

    @pl.when(i == 0)
    def _init():
        loss_ref[...] = jnp.zeros_like(loss_ref)

    x0 = x_ref[...]
    H = BLK // NSPLIT
    rs = [x0[h * H:(h + 1) * H] for h in range(NSPLIT)]
    scale = 1.0 / (n_tokens * D)
    idx_cols = [[] for _ in range(NSPLIT)]
    losses = []
    for q in range(NQ):
        cb = cb_ref[q]
        cb16 = cb.astype(jnp.bfloat16)
        c2 = jnp.transpose(jnp.sum(cb * cb, axis=1, keepdims=True))
        rem = cb - cb16.astype(jnp.float32)
        cb_mid = rem.astype(jnp.bfloat16)
        cb_lo = (rem - cb_mid.astype(jnp.float32)).astype(jnp.bfloat16)
        gain = jnp.concatenate([cb16, cb_mid, cb_lo], axis=1)
        lsum = 0.0
        for h in range(NSPLIT):
            r = rs[h]
            dots = jax.lax.dot_general(
                r.astype(jnp.bfloat16), cb16,
                (((1,), (1,)), ((), ())),
                preferred_element_type=jnp.float32)
            r2 = jnp.sum(r * r, axis=1, keepdims=True)
            scores = (r2 - 2.0 * dots) + c2
            m = jnp.min(scores, axis=1, keepdims=True)
            iota = jax.lax.broadcasted_iota(jnp.int32, (H, K), 1)
            idxfull = jnp.where(scores == m, iota, K)
            idx = jnp.min(idxfull, axis=1).astype(jnp.int32)
            oh = (idxfull == idx[:, None]).astype(jnp.bfloat16)
            q3 = jax.lax.dot_general(
                oh, gain, (((1,), (0,)), ((), ())),
                preferred_element_type=jnp.float32)
            quant = (q3[:, :D] + q3[:, D:2 * D]) + q3[:, 2 * D:]
            r = r - quant
            rs[h] = r
            lsum = lsum + jnp.sum(r * r)
            idx_cols[h].append(idx[:, None])
        losses.append(lsum * scale)
    qout_ref[...] = x0 - jnp.concatenate(rs, axis=0)
    idx_ref[...] = jnp.concatenate(
        [jnp.concatenate(cols, axis=1) for cols in idx_cols], axis=0)
    loss_ref[...] += jnp.stack(losses)[None, :]


def kernel(x, codebooks):
    B, N, D_ = x.shape
    n_tokens = B * N
    xf = x.reshape(n_tokens, D_)
    body = functools.partial(_rvq_kernel, n_tokens=n_tokens)
    qout, idx, loss = pl.pallas_call(
        body,
        grid=(n_tokens // BLK,),
        in_specs=[
            pl.BlockSpec((BLK, D), lambda i: (i, 0)),
            pl.BlockSpec((NQ, K, D), lambda i: (0, 0, 0)),
        ],
        out_specs=[
            pl.BlockSpec((BLK, D), lambda i: (i, 0)),
            pl.BlockSpec((BLK, NQ), lambda i: (i, 0)),
            pl.BlockSpec((1, NQ), lambda i: (0, 0)),
        ],
        out_shape=[
            jax.ShapeDtypeStruct((n_tokens, D_), jnp.float32),
            jax.ShapeDtypeStruct((n_tokens, NQ), jnp.int32),
            jax.ShapeDtypeStruct((1, NQ), jnp.float32),
        ],
    )(xf, codebooks)
    return (qout.reshape(B, N, D_), idx.reshape(B, N, NQ), loss.reshape(NQ))


# speedup vs baseline: 1.0723x; 1.0723x over previous
import functools

import jax
import jax.numpy as jnp
from jax.experimental import pallas as pl
from jax.experimental.pallas import tpu as pltpu

NQ = 8
K = 1024
D = 64
BLK = 4096
NSPLIT = 1


def _rvq_kernel(x_ref, cb_ref, qout_ref, idx_ref, loss_ref, c2_s, gain_s,
                *, n_tokens):
    i = pl.program_id(0)

    @pl.when(i == 0)
    def _init():
        loss_ref[...] = jnp.zeros_like(loss_ref)
        for q in range(NQ):
            cb = cb_ref[q]
            cb16 = cb.astype(jnp.bfloat16)
            rem = cb - cb16.astype(jnp.float32)
            cb_mid = rem.astype(jnp.bfloat16)
            cb_lo = (rem - cb_mid.astype(jnp.float32)).astype(jnp.bfloat16)
            gain_s[q] = jnp.concatenate([cb16, cb_mid, cb_lo], axis=1)
            c2_s[q:q + 1, :] = jnp.transpose(
                jnp.sum(cb * cb, axis=1, keepdims=True))

    x0 = x_ref[...]
    H = BLK // NSPLIT
    rs = [x0[h * H:(h + 1) * H] for h in range(NSPLIT)]
    scale = 1.0 / (n_tokens * D)
    idx_cols = [[] for _ in range(NSPLIT)]
    losses = []
    for q in range(NQ):
        cb16 = gain_s[q, :, :D]
        c2 = c2_s[q:q + 1, :]
        gain = gain_s[q]
        lsum = 0.0
        for h in range(NSPLIT):
            r = rs[h]
            dots = jax.lax.dot_general(
                r.astype(jnp.bfloat16), cb16,
                (((1,), (1,)), ((), ())),
                preferred_element_type=jnp.float32)
            r2 = jnp.sum(r * r, axis=1, keepdims=True)
            scores = (r2 - 2.0 * dots) + c2
            idx = jnp.argmin(scores, axis=1).astype(jnp.int32)
            oh = (jax.lax.broadcasted_iota(jnp.int32, (H, K), 1)
                  == idx[:, None]).astype(jnp.bfloat16)
            q3 = jax.lax.dot_general(
                oh, gain, (((1,), (0,)), ((), ())),
                preferred_element_type=jnp.float32)
            quant = (q3[:, :D] + q3[:, D:2 * D]) + q3[:, 2 * D:]
            r = r - quant
            rs[h] = r
            lsum = lsum + jnp.sum(r * r)
            idx_cols[h].append(idx[:, None])
        losses.append(lsum * scale)
    qout_ref[...] = x0 - jnp.concatenate(rs, axis=0)
    idx_ref[...] = jnp.concatenate(
        [jnp.concatenate(cols, axis=1) for cols in idx_cols], axis=0)
    loss_ref[...] += jnp.stack(losses)[None, :]


def kernel(x, codebooks):
    B, N, D_ = x.shape
    n_tokens = B * N
    xf = x.reshape(n_tokens, D_)
    body = functools.partial(_rvq_kernel, n_tokens=n_tokens)
    qout, idx, loss = pl.pallas_call(
        body,
        grid=(n_tokens // BLK,),
        in_specs=[
            pl.BlockSpec((BLK, D), lambda i: (i, 0)),
            pl.BlockSpec((NQ, K, D), lambda i: (0, 0, 0)),
        ],
        out_specs=[
            pl.BlockSpec((BLK, D), lambda i: (i, 0)),
            pl.BlockSpec((BLK, NQ), lambda i: (i, 0)),
            pl.BlockSpec((1, NQ), lambda i: (0, 0)),
        ],
        out_shape=[
            jax.ShapeDtypeStruct((n_tokens, D_), jnp.float32),
            jax.ShapeDtypeStruct((n_tokens, NQ), jnp.int32),
            jax.ShapeDtypeStruct((1, NQ), jnp.float32),
        ],
        scratch_shapes=[
            pltpu.VMEM((NQ, K), jnp.float32),
            pltpu.VMEM((NQ, K, 3 * D), jnp.bfloat16),
        ],
    )(xf, codebooks)
    return (qout.reshape(B, N, D_), idx.reshape(B, N, NQ), loss.reshape(NQ))
